# baseline (speedup 1.0000x reference)
import functools

import jax
import jax.numpy as jnp
import numpy as np
from jax import lax
from jax.experimental import pallas as pl
from jax.experimental.pallas import tpu as pltpu
from jax.experimental.pallas import tpu_sc as plsc

N = 10000
NSP = 10240
NPW = 320
K = 64
NCX = 44
NCELLS = NCX * NCX
CSP = 2048
NC = 2
FAR = 1.0e6
CC = 512
BLKR = 256
NCHK = 16
_INV_PI = float(1.0 / np.pi)

_mesh = functools.partial(plsc.VectorSubcoreMesh,
                          core_axis_name="c", subcore_axis_name="s")
_CP = pltpu.CompilerParams(needs_layout_passes=False)
_CPU = pltpu.CompilerParams(needs_layout_passes=False,
                            use_tc_tiling_on_sc=False)


def _wid():
    return lax.axis_index("s") * NC + lax.axis_index("c")


def _i16(v):
    return jnp.full((16,), v, jnp.int32)


def _lane0():
    return lax.iota(jnp.int32, 16) == 0


def _sort_body(posf_hbm, perm_hbm, sinv_hbm, cs_hbm, psf_hbm,
               pos_v, cid_v, hist_v, cur_v, cs_v, perm_v, sinv_v, ps_v):
    w = _wid()

    @pl.when(w == 0)
    def _():
        pltpu.sync_copy(posf_hbm, pos_v)
        iota = lax.iota(jnp.int32, 16)
        l0 = _lane0()
        ones = jnp.full((16,), 1, jnp.int32)

        def cell_chunk(k, _):
            idx = k * 16 + iota
            idxc = jnp.minimum(idx, N - 1)
            px = plsc.load_gather(pos_v, [idxc * 2])
            py = plsc.load_gather(pos_v, [idxc * 2 + 1])
            cx = jnp.clip((px * NCX).astype(jnp.int32), 0, NCX - 1)
            cy = jnp.clip((py * NCX).astype(jnp.int32), 0, NCX - 1)
            cid_v[pl.ds(k * 16, 16)] = cy * NCX + cx
            return 0

        lax.fori_loop(0, N // 16, cell_chunk, 0)

        def zero_chunk(k, _):
            hist_v[pl.ds(k * 16, 16)] = jnp.zeros((16,), jnp.int32)
            return 0

        lax.fori_loop(0, CSP // 16, zero_chunk, 0)

        def hist_step(i, _):
            c = plsc.load_gather(cid_v, [_i16(i)])
            plsc.addupdate_scatter(hist_v, [c], ones, mask=l0)
            return 0

        lax.fori_loop(0, N, hist_step, 0)

        def prefix_step(c, run):
            cv = _i16(c)
            plsc.store_scatter(cs_v, [cv], run, mask=l0)
            plsc.store_scatter(cur_v, [cv], run, mask=l0)
            return run + plsc.load_gather(hist_v, [cv])

        total = lax.fori_loop(0, NCELLS, prefix_step,
                              jnp.zeros((16,), jnp.int32))

        def fill_cs(c, _):
            plsc.store_scatter(cs_v, [_i16(c)], total, mask=l0)
            return 0

        lax.fori_loop(NCELLS, CSP, fill_cs, 0)

        def scat_step(i, _):
            c = plsc.load_gather(cid_v, [_i16(i)])
            s = plsc.load_gather(cur_v, [c])
            plsc.store_scatter(perm_v, [s], _i16(i), mask=l0)
            plsc.store_scatter(cur_v, [c], s + 1, mask=l0)
            return 0

        lax.fori_loop(0, N, scat_step, 0)

        def pad_perm(k, _):
            perm_v[pl.ds(N + k * 16, 16)] = N + k * 16 + iota
            return 0

        lax.fori_loop(0, (NSP - N) // 16, pad_perm, 0)

        def inv_chunk(k, _):
            pv = perm_v[pl.ds(k * 16, 16)]
            plsc.store_scatter(sinv_v, [pv], k * 16 + iota)
            return 0

        lax.fori_loop(0, NSP // 16, inv_chunk, 0)

        def ps_chunk(k, _):
            sidx = k * 16 + iota
            pv = perm_v[pl.ds(k * 16, 16)]
            pvc = jnp.minimum(pv, N - 1)
            gx = plsc.load_gather(pos_v, [pvc * 2])
            gy = plsc.load_gather(pos_v, [pvc * 2 + 1])
            plsc.store_scatter(ps_v, [sidx * 2], gx)
            plsc.store_scatter(ps_v, [sidx * 2 + 1], gy)
            return 0

        lax.fori_loop(0, NSP // 16, ps_chunk, 0)

        def ps_pad(k, _):
            ps_v[pl.ds(2 * N + k * 16, 16)] = jnp.full((16,), FAR,
                                                       jnp.float32)
            return 0

        lax.fori_loop(0, 2 * (NSP - N) // 16, ps_pad, 0)

        pltpu.sync_copy(perm_v, perm_hbm)
        pltpu.sync_copy(sinv_v, sinv_hbm)
        pltpu.sync_copy(cs_v, cs_hbm)
        pltpu.sync_copy(ps_v, psf_hbm)


def sc_sort(posf):
    f = pl.kernel(
        _sort_body,
        out_type=[
            jax.ShapeDtypeStruct((NSP,), jnp.int32),
            jax.ShapeDtypeStruct((NSP,), jnp.int32),
            jax.ShapeDtypeStruct((CSP,), jnp.int32),
            jax.ShapeDtypeStruct((2 * NSP,), jnp.float32),
        ],
        mesh=_mesh(),
        compiler_params=_CP,
        scratch_types=[
            pltpu.VMEM((2 * N,), jnp.float32),
            pltpu.VMEM((NSP,), jnp.int32),
            pltpu.VMEM((CSP,), jnp.int32),
            pltpu.VMEM((CSP,), jnp.int32),
            pltpu.VMEM((CSP,), jnp.int32),
            pltpu.VMEM((NSP,), jnp.int32),
            pltpu.VMEM((NSP,), jnp.int32),
            pltpu.VMEM((2 * NSP,), jnp.float32),
        ],
    )
    return f(posf)


def _nbr_body(psf_hbm, cs_hbm, sup_hbm, nbr_hbm, cnt_hbm, dx_hbm, dy_hbm,
              pos_v, cs_v, sup_v, nbr_v, cnt_v, dx_v, dy_v):
    w = _wid()
    base = w * NPW
    pltpu.sync_copy(psf_hbm, pos_v)
    pltpu.sync_copy(cs_hbm, cs_v)
    pltpu.sync_copy(sup_hbm, sup_v)
    iota = lax.iota(jnp.int32, 16)
    l0 = _lane0()
    supv = sup_v[pl.ds(0, 16)]
    r2 = (supv * supv)[0]

    def zero_chunk(k, _):
        nbr_v[pl.ds(k * 16, 16)] = jnp.zeros((16,), jnp.int32)
        return 0

    lax.fori_loop(0, (NPW * K + 32) // 16, zero_chunk, 0)

    def node(n, _):
        s = base + n
        sv = _i16(s)
        qxv = plsc.load_gather(pos_v, [sv * 2])
        qyv = plsc.load_gather(pos_v, [sv * 2 + 1])
        qx = qxv[0]
        qy = qyv[0]
        cx = jnp.clip((qxv * NCX).astype(jnp.int32), 0, NCX - 1)[0]
        cy = jnp.clip((qyv * NCX).astype(jnp.int32), 0, NCX - 1)[0]
        lo = jnp.maximum(cx - 1, 0)
        hi = jnp.minimum(cx + 1, NCX - 1)
        cnt = jnp.int32(0)
        for dr in (-1, 0, 1):
            row = cy + dr
            ok = jnp.logical_and(row >= 0, row < NCX)
            rowc = jnp.clip(row, 0, NCX - 1)
            a = plsc.load_gather(cs_v, [_i16(rowc * NCX + lo)])[0]
            b = plsc.load_gather(cs_v, [_i16(rowc * NCX + hi + 1)])[0]
            a = jnp.where(ok, a, 0)
            b = jnp.where(ok, b, 0)

            def cond(st):
                return jnp.logical_and(st[0] < st[1], st[2] < K - 15)

            def step(st):
                j0, bb, cc = st
                idx = j0 + iota
                idxc = jnp.minimum(idx, NSP - 1)
                px = plsc.load_gather(pos_v, [idxc * 2])
                py = plsc.load_gather(pos_v, [idxc * 2 + 1])
                dx = qx - px
                dy = qy - py
                d2 = dx * dx + dy * dy
                keep = (idx < bb) & (d2 < r2) & (idx != s)
                off = n * K + cc
                plsc.store_compressed(nbr_v.at[pl.ds(off, 16)], idx,
                                      mask=keep)
                plsc.store_compressed(dx_v.at[pl.ds(off, 16)], dx, mask=keep)
                plsc.store_compressed(dy_v.at[pl.ds(off, 16)], dy, mask=keep)
                cc = cc + jnp.sum(keep.astype(jnp.int32))
                return j0 + 16, bb, cc

            _, _, cnt = lax.while_loop(cond, step, (a, b, cnt))
        plsc.store_scatter(cnt_v, [_i16(n)], _i16(cnt), mask=l0)
        return 0

    lax.fori_loop(0, NPW, node, 0)

    pltpu.sync_copy(nbr_v.at[pl.ds(0, NPW * K)],
                    nbr_hbm.at[pl.ds(base * K, NPW * K)])
    pltpu.sync_copy(dx_v.at[pl.ds(0, NPW * K)],
                    dx_hbm.at[pl.ds(base * K, NPW * K)])
    pltpu.sync_copy(dy_v.at[pl.ds(0, NPW * K)],
                    dy_hbm.at[pl.ds(base * K, NPW * K)])
    pltpu.sync_copy(cnt_v, cnt_hbm.at[pl.ds(base, NPW)])


def sc_nbr(psf, cell_start, support):
    sup = jnp.broadcast_to(jnp.reshape(support, (1,)), (16,))
    f = pl.kernel(
        _nbr_body,
        out_type=[
            jax.ShapeDtypeStruct((NSP * K,), jnp.int32),
            jax.ShapeDtypeStruct((NSP,), jnp.int32),
            jax.ShapeDtypeStruct((NSP * K,), jnp.float32),
            jax.ShapeDtypeStruct((NSP * K,), jnp.float32),
        ],
        mesh=_mesh(),
        compiler_params=_CP,
        scratch_types=[
            pltpu.VMEM((2 * NSP,), jnp.float32),
            pltpu.VMEM((CSP,), jnp.int32),
            pltpu.VMEM((16,), jnp.float32),
            pltpu.VMEM((NPW * K + 32,), jnp.int32),
            pltpu.VMEM((NPW,), jnp.int32),
            pltpu.VMEM((NPW * K + 32,), jnp.float32),
            pltpu.VMEM((NPW * K + 32,), jnp.float32),
        ],
    )
    return f(psf, cell_start, sup)


def _agg_body(x_hbm, nbr_hbm, cnt_hbm, klw_hbm, klp_hbm, g_hbm,
              nbr_v, cnt_v, klw_v, klp_v, xg_v, acc_v, sem, *, cin, act):
    w = _wid()
    ncc = cin // 16
    iota = lax.iota(jnp.int32, 16)
    tk = jnp.minimum(iota, 3) * K

    def chunk(c, _):
        n0 = w * NPW + c * NCHK
        pltpu.sync_copy(nbr_hbm.at[pl.ds(n0 * K, NCHK * K)], nbr_v)
        pltpu.sync_copy(cnt_hbm.at[pl.ds(n0, NCHK)], cnt_v)
        pltpu.sync_copy(klw_hbm.at[pl.ds(n0 * 4 * K, NCHK * 4 * K)], klw_v)
        pltpu.sync_copy(klp_hbm.at[pl.ds(n0 * 4 * K, NCHK * 4 * K)], klp_v)
        pltpu.async_copy(x_hbm.at[nbr_v], xg_v, sem).wait()

        def zero(k, _):
            acc_v[pl.ds(k * 16, 16)] = jnp.zeros((16,), jnp.float32)
            return 0

        lax.fori_loop(0, NCHK * cin, zero, 0)

        cntv = cnt_v[pl.ds(0, 16)]
        for nl in range(NCHK):
            cn = cntv[nl]

            def edge(k, _, nl=nl):
                row = nl * K + k
                xrow = [xg_v[row, pl.ds(cc * 16, 16)] for cc in range(ncc)]
                if act:
                    xrow = [jnp.maximum(xv, 0.0) for xv in xrow]
                kw = plsc.load_gather(klw_v, [_i16(nl * 4 * K + k) + tk])
                kp = plsc.load_gather(klp_v, [_i16(nl * 4 * K + k) + tk])
                for t in range(4):
                    v = kw[t]
                    off = nl * (16 * cin) + kp[t] * cin
                    for cc in range(ncc):
                        plsc.addupdate(acc_v.at[pl.ds(off + cc * 16, 16)],
                                       v * xrow[cc])
                return 0

            lax.fori_loop(0, cn, edge, 0)
        pltpu.sync_copy(acc_v, g_hbm.at[pl.ds(n0 * 16 * cin, NCHK * 16 * cin)])
        return 0

    lax.fori_loop(0, NPW // NCHK, chunk, 0)


def sc_agg(x, nbrf, cnt, klw, klp, cin, act=False):
    f = pl.kernel(
        functools.partial(_agg_body, cin=cin, act=act),
        out_type=[jax.ShapeDtypeStruct((NSP * 16 * cin,), jnp.float32)],
        mesh=_mesh(),
        compiler_params=_CPU,
        scratch_types=[
            pltpu.VMEM((NCHK * K,), jnp.int32),
            pltpu.VMEM((16,), jnp.int32),
            pltpu.VMEM((NCHK * 4 * K,), jnp.float32),
            pltpu.VMEM((NCHK * 4 * K,), jnp.int32),
            pltpu.VMEM((NCHK * K, cin), jnp.float32),
            pltpu.VMEM((NCHK * 16 * cin,), jnp.float32),
            pltpu.SemaphoreType.DMA,
        ],
    )
    return f(x, nbrf, cnt, klw, klp)[0].reshape(NSP, 16 * cin)


def _rowg_body(x_hbm, idx_hbm, o_hbm, idx_v, xg_v, sem):
    w = _wid()
    base = w * NPW
    pltpu.sync_copy(idx_hbm.at[pl.ds(base, NPW)], idx_v)
    pltpu.async_copy(x_hbm.at[idx_v], xg_v, sem).wait()
    pltpu.sync_copy(xg_v, o_hbm.at[pl.ds(base, NPW)])


def sc_gather_rows(x, idx):
    d = x.shape[1]
    f = pl.kernel(
        _rowg_body,
        out_type=[jax.ShapeDtypeStruct((NSP, d), jnp.float32)],
        mesh=_mesh(),
        compiler_params=_CPU,
        scratch_types=[
            pltpu.VMEM((NPW,), jnp.int32),
            pltpu.VMEM((NPW, d), jnp.float32),
            pltpu.SemaphoreType.DMA,
        ],
    )
    return f(x, idx)[0]


def _kl_body(s_ref, dx_ref, dy_ref, cnt_ref, klw_ref, klp_ref):
    s = s_ref[0, 0]
    inv = 1.0 / s
    dx = dx_ref[...]
    dy = dy_ref[...]
    ex = jnp.clip(dx * inv, -1.0, 1.0)
    ey = jnp.clip(dy * inv, -1.0, 1.0)
    r = jnp.sqrt(ex * ex + ey * ey)
    th = jnp.arctan2(ey, ex) * _INV_PI
    fu = r * 3.0
    u0 = jnp.clip(fu.astype(jnp.int32), 0, 2)
    au = fu - u0.astype(jnp.float32)
    fv = (th + 1.0) * 1.5
    v0 = jnp.clip(fv.astype(jnp.int32), 0, 2)
    av = fv - v0.astype(jnp.float32)
    valid = (jax.lax.broadcasted_iota(jnp.int32, (BLKR, K), 1)
             < cnt_ref[...])
    wu0 = jnp.where(valid, 1.0 - au, 0.0)
    wu1 = jnp.where(valid, au, 0.0)
    av = jnp.where(valid, av, 0.0)
    p00 = jnp.where(valid, u0 * 4 + v0, 0)
    klw_ref[:, 0 * K:1 * K] = wu0 * (1.0 - av)
    klp_ref[:, 0 * K:1 * K] = p00
    klw_ref[:, 1 * K:2 * K] = wu0 * av
    klp_ref[:, 1 * K:2 * K] = p00 + 1
    klw_ref[:, 2 * K:3 * K] = wu1 * (1.0 - av)
    klp_ref[:, 2 * K:3 * K] = p00 + 4
    klw_ref[:, 3 * K:4 * K] = wu1 * av
    klp_ref[:, 3 * K:4 * K] = p00 + 5


def klgen(support, dx, dy, cnt):
    s = jnp.reshape(support, (1, 1))
    return pl.pallas_call(
        _kl_body,
        grid=(NSP // BLKR,),
        in_specs=[
            pl.BlockSpec((1, 1), lambda i: (0, 0)),
            pl.BlockSpec((BLKR, K), lambda i: (i, 0)),
            pl.BlockSpec((BLKR, K), lambda i: (i, 0)),
            pl.BlockSpec((BLKR, 1), lambda i: (i, 0)),
        ],
        out_specs=[
            pl.BlockSpec((BLKR, 4 * K), lambda i: (i, 0)),
            pl.BlockSpec((BLKR, 4 * K), lambda i: (i, 0)),
        ],
        out_shape=[
            jax.ShapeDtypeStruct((NSP, 4 * K), jnp.float32),
            jax.ShapeDtypeStruct((NSP, 4 * K), jnp.int32),
        ],
    )(s, dx, dy, cnt)


def _mm_body(a_ref, b_ref, bias_ref, *rest, act, res, scale):
    if res:
        r_ref, o_ref = rest
    else:
        (o_ref,) = rest
    a = a_ref[...]
    if act:
        a = jnp.maximum(a, 0.0)
    o = jnp.dot(a, b_ref[...], preferred_element_type=jnp.float32) + bias_ref[...]
    if res:
        o = o + r_ref[...]
    o_ref[...] = o * scale


def _mm(a, b, bias, act=False, res=None, scale=1.0):
    m, k = a.shape
    _, n = b.shape
    inputs = [a, b, bias.reshape(1, n)]
    specs = [
        pl.BlockSpec((CC, k), lambda i: (i, 0)),
        pl.BlockSpec((k, n), lambda i: (0, 0)),
        pl.BlockSpec((1, n), lambda i: (0, 0)),
    ]
    if res is not None:
        inputs.append(res)
        specs.append(pl.BlockSpec((CC, n), lambda i: (i, 0)))
    return pl.pallas_call(
        functools.partial(_mm_body, act=act, res=res is not None, scale=scale),
        grid=(m // CC,),
        in_specs=specs,
        out_specs=pl.BlockSpec((CC, n), lambda i: (i, 0)),
        out_shape=jax.ShapeDtypeStruct((m, n), jnp.float32),
    )(*inputs)


def _wflat(W, cin_pad=None, cout_pad=None):
    nb, mb, cin, cout = W.shape
    Wf = W.reshape(nb * mb, cin, cout)
    if cin_pad is not None and cin_pad > cin:
        Wf = jnp.concatenate(
            [Wf, jnp.zeros((nb * mb, cin_pad - cin, cout), jnp.float32)],
            axis=1)
        cin = cin_pad
    if cout_pad is not None and cout_pad > cout:
        Wf = jnp.concatenate(
            [Wf, jnp.zeros((nb * mb, cin, cout_pad - cout), jnp.float32)],
            axis=2)
        cout = cout_pad
    return Wf.reshape(nb * mb * cin, cout)


def kernel(fluidPositions, boundaryPositions, fluidFeatures, boundaryFeatures,
           support, W0, b0, W1, b1, W2, b2, W3, b3, fcW0, fcb0, fcW1, fcb1,
           fcW2, fcb2, fcW3, fcb3):
    posf = jnp.reshape(fluidPositions, (2 * N,))
    perm, sinv, cs, psf = sc_sort(posf)
    nbrf, cnt, dxe, dye = sc_nbr(psf, cs, support)
    klw, klp = klgen(support, dxe.reshape(NSP, K), dye.reshape(NSP, K),
                     cnt.reshape(NSP, 1))
    klwf = klw.reshape(NSP * 4 * K)
    klpf = klp.reshape(NSP * 4 * K)

    x0 = jnp.concatenate(
        [fluidFeatures, jnp.zeros((N, 8), jnp.float32)], axis=1)
    x0 = jnp.concatenate([x0, jnp.zeros((NSP - N, 16), jnp.float32)], axis=0)
    xs0 = sc_gather_rows(x0, perm)

    fcw0 = jnp.concatenate([fcW0.T, jnp.zeros((8, 32), jnp.float32)], axis=0)
    lin0 = _mm(xs0, fcw0, fcb0)
    g0 = sc_agg(xs0, nbrf, cnt, klwf, klpf, cin=16)
    conv0 = _mm(g0, _wflat(W0, cin_pad=16), b0)
    ans0 = jnp.concatenate([lin0, conv0], axis=1)

    lin1 = _mm(ans0, fcW1.T, fcb1, act=True)
    g1 = sc_agg(ans0, nbrf, cnt, klwf, klpf, cin=64, act=True)
    ans1 = _mm(g1, _wflat(W1), b1, res=lin1)

    lin2 = _mm(ans1, fcW2.T, fcb2, act=True, res=ans1)
    g2 = sc_agg(ans1, nbrf, cnt, klwf, klpf, cin=32, act=True)
    ans2 = _mm(g2, _wflat(W2), b2, res=lin2)

    fcw3 = jnp.concatenate([fcW3.T, jnp.zeros((32, 14), jnp.float32)], axis=1)
    fcb3p = jnp.concatenate([fcb3, jnp.zeros((14,), jnp.float32)])
    b3p = jnp.concatenate([b3, jnp.zeros((14,), jnp.float32)])
    lin3 = _mm(ans2, fcw3, fcb3p, act=True)
    g3 = sc_agg(ans2, nbrf, cnt, klwf, klpf, cin=32, act=True)
    out_s = _mm(g3, _wflat(W3, cout_pad=16), b3p, res=lin3, scale=1.0 / 128.0)

    out = sc_gather_rows(out_s, sinv)
    return out[:N, :2]
